# baseline (speedup 1.0000x reference)
import functools

import jax
import jax.numpy as jnp
from jax import lax
from jax.experimental import pallas as pl
from jax.experimental.pallas import tpu as pltpu
from jax.experimental.pallas import tpu_sc as plsc

_N = 10000
_K = 32
_D = 128
_CB = 8
_NW = 32
_BASE = 312
_MAXC = 40
_DG = _D // 16


def _sc_body(ce_hbm, al_hbm, msg_hbm, out_hbm, msg_v, al_v, ce_v, out_all,
             sem0, sem1):
    info = plsc.get_sparse_core_info()
    nc = info.num_cores
    wid = lax.axis_index("s") * nc + lax.axis_index("c")
    w_start = _BASE * wid + _CB * jnp.minimum(wid, 2)
    n_chunks = jnp.where(wid < 2, _MAXC, _MAXC - 1)
    sems = (sem0, sem1)

    def chunk_start(c):
        return w_start + jnp.minimum(c, n_chunks - 1) * _CB

    def issue(c, slot):
        st = chunk_start(c)
        sem = sems[slot]
        pltpu.async_copy(msg_hbm.at[pl.ds(st, _CB)], msg_v.at[slot], sem)
        pltpu.async_copy(al_hbm.at[pl.ds(st, _CB)], al_v.at[slot], sem)
        pltpu.async_copy(
            ce_hbm.at[pl.ds(st, _CB), pl.ds(0, 1)], ce_v.at[slot], sem)

    def drain(c, slot):
        st = chunk_start(c)
        sem = sems[slot]
        pltpu.make_async_copy(
            msg_hbm.at[pl.ds(st, _CB)], msg_v.at[slot], sem).wait()
        pltpu.make_async_copy(
            al_hbm.at[pl.ds(st, _CB)], al_v.at[slot], sem).wait()
        pltpu.make_async_copy(
            ce_hbm.at[pl.ds(st, _CB), pl.ds(0, 1)], ce_v.at[slot], sem).wait()

    def compute(c, slot):
        row0 = jnp.minimum(c, n_chunks - 1) * _CB

        def ibody(i, carry):
            accs = tuple(
                ce_v[slot, i, 0, pl.ds(g * 16, 16)] for g in range(_DG))

            def kgbody(kg, accs):
                av = al_v[slot, i, pl.ds(kg * 16, 16)]
                for j in range(16):
                    a = av[j]
                    k = kg * 16 + j
                    accs = tuple(
                        accs[g] + a * msg_v[slot, i, k, pl.ds(g * 16, 16)]
                        for g in range(_DG))
                return accs

            accs = lax.fori_loop(0, _K // 16, kgbody, accs)
            for g in range(_DG):
                out_all[row0 + i, pl.ds(g * 16, 16)] = accs[g]
            return carry

        lax.fori_loop(0, _CB, ibody, 0)

    issue(0, 0)

    def loop_body(c2, carry):
        for b in (0, 1):
            cur = 2 * c2 + b
            issue(cur + 1, b ^ 1)
            drain(cur, b)
            compute(cur, b)
        return carry

    lax.fori_loop(0, _MAXC // 2, loop_body, 0)
    drain(_MAXC, 0)

    pltpu.sync_copy(out_all.at[pl.ds(0, _BASE)],
                    out_hbm.at[pl.ds(w_start, _BASE)])
    tail = jnp.where(wid < 2, _BASE, _BASE - _CB)
    pltpu.sync_copy(out_all.at[pl.ds(tail, _CB)],
                    out_hbm.at[pl.ds(w_start + tail, _CB)])


def kernel(curr_emb, alpha, msg):
    mesh = plsc.VectorSubcoreMesh(core_axis_name="c", subcore_axis_name="s")
    run = functools.partial(
        pl.kernel,
        mesh=mesh,
        out_type=jax.ShapeDtypeStruct((_N, _D), jnp.float32),
        scratch_types=[
            pltpu.VMEM((2, _CB, _K, _D), jnp.float32),
            pltpu.VMEM((2, _CB, _K), jnp.float32),
            pltpu.VMEM((2, _CB, 1, _D), jnp.float32),
            pltpu.VMEM((_MAXC * _CB, _D), jnp.float32),
            pltpu.SemaphoreType.DMA,
            pltpu.SemaphoreType.DMA,
        ],
    )(_sc_body)
    return run(curr_emb, alpha[:, :, 0], msg)


# speedup vs baseline: 1.0353x; 1.0353x over previous
import functools

import jax
import jax.numpy as jnp
from jax import lax
from jax.experimental import pallas as pl
from jax.experimental.pallas import tpu as pltpu
from jax.experimental.pallas import tpu_sc as plsc

_N = 10000
_K = 32
_D = 128
_NT = 3600
_BN = 400
_CB = 8
_NW = 32
_PW = (_N - _NT) // _NW
_NCH = _PW // _CB
_DG = _D // 16


def _tc_body(ce_ref, al_ref, msg_ref, out_ref):
    a = al_ref[...]
    m = msg_ref[...]
    w = m * a[:, :, None]
    w = w[:, :16, :] + w[:, 16:, :]
    w = w[:, :8, :] + w[:, 8:, :]
    acc = jnp.sum(w, axis=1)
    out_ref[...] = ce_ref[...] + acc


def _sc_body(ce_hbm, al_hbm, msg_hbm, out_hbm, msg_v, al_v, ce_v, out_all,
             sem0, sem1):
    info = plsc.get_sparse_core_info()
    nc = info.num_cores
    wid = lax.axis_index("s") * nc + lax.axis_index("c")
    o_start = _PW * wid
    n_start = _NT + o_start
    sems = (sem0, sem1)

    def chunk_off(c):
        return jnp.minimum(c, _NCH - 1) * _CB

    def issue(c, slot):
        st = n_start + chunk_off(c)
        sem = sems[slot]
        pltpu.async_copy(msg_hbm.at[pl.ds(st, _CB)], msg_v.at[slot], sem)
        pltpu.async_copy(al_hbm.at[pl.ds(st, _CB)], al_v.at[slot], sem)
        pltpu.async_copy(
            ce_hbm.at[pl.ds(st, _CB), pl.ds(0, 1)], ce_v.at[slot], sem)

    def drain(c, slot):
        st = n_start + chunk_off(c)
        sem = sems[slot]
        pltpu.make_async_copy(
            msg_hbm.at[pl.ds(st, _CB)], msg_v.at[slot], sem).wait()
        pltpu.make_async_copy(
            al_hbm.at[pl.ds(st, _CB)], al_v.at[slot], sem).wait()
        pltpu.make_async_copy(
            ce_hbm.at[pl.ds(st, _CB), pl.ds(0, 1)], ce_v.at[slot], sem).wait()

    def compute(c, slot):
        row0 = chunk_off(c)

        def ibody(i, carry):
            accs = tuple(
                ce_v[slot, i, 0, pl.ds(g * 16, 16)] for g in range(_DG))

            def kgbody(kg, accs):
                av = al_v[slot, i, pl.ds(kg * 16, 16)]
                for j in range(16):
                    a = av[j]
                    k = kg * 16 + j
                    accs = tuple(
                        accs[g] + a * msg_v[slot, i, k, pl.ds(g * 16, 16)]
                        for g in range(_DG))
                return accs

            accs = lax.fori_loop(0, _K // 16, kgbody, accs)
            for g in range(_DG):
                out_all[row0 + i, pl.ds(g * 16, 16)] = accs[g]
            return carry

        lax.fori_loop(0, _CB, ibody, 0)

    issue(0, 0)
    nloop = (_NCH + 1) // 2

    def loop_body(c2, carry):
        for b in (0, 1):
            cur = 2 * c2 + b
            issue(cur + 1, b ^ 1)
            drain(cur, b)
            compute(cur, b)
        return carry

    lax.fori_loop(0, nloop, loop_body, 0)
    drain(2 * nloop, 0)

    pltpu.sync_copy(out_all, out_hbm.at[pl.ds(o_start, _PW)])


def kernel(curr_emb, alpha, msg):
    al = alpha[:, :, 0]
    ce_tc = curr_emb[:_NT, 0, :]
    grid = (_NT // _BN,)
    out_tc = pl.pallas_call(
        _tc_body,
        grid=grid,
        in_specs=[
            pl.BlockSpec((_BN, _D), lambda i: (i, 0)),
            pl.BlockSpec((_BN, _K), lambda i: (i, 0)),
            pl.BlockSpec((_BN, _K, _D), lambda i: (i, 0, 0)),
        ],
        out_specs=pl.BlockSpec((_BN, _D), lambda i: (i, 0)),
        out_shape=jax.ShapeDtypeStruct((_NT, _D), jnp.float32),
        compiler_params=pltpu.CompilerParams(
            dimension_semantics=("parallel",),
        ),
    )(ce_tc, al, msg)

    mesh = plsc.VectorSubcoreMesh(core_axis_name="c", subcore_axis_name="s")
    out_sc = functools.partial(
        pl.kernel,
        mesh=mesh,
        out_type=jax.ShapeDtypeStruct((_N - _NT, _D), jnp.float32),
        scratch_types=[
            pltpu.VMEM((2, _CB, _K, _D), jnp.float32),
            pltpu.VMEM((2, _CB, _K), jnp.float32),
            pltpu.VMEM((2, _CB, 1, _D), jnp.float32),
            pltpu.VMEM((_PW, _D), jnp.float32),
            pltpu.SemaphoreType.DMA,
            pltpu.SemaphoreType.DMA,
        ],
    )(_sc_body)(curr_emb, al, msg)

    return jnp.concatenate([out_tc, out_sc], axis=0)
